# baseline (speedup 1.0000x reference)
import functools

import jax
import jax.numpy as jnp
from jax import lax
from jax.experimental import pallas as pl
from jax.experimental.pallas import tpu as pltpu
from jax.experimental.pallas import tpu_sc as plsc

B = 1024
N_PER = 32
H = 128
L = 16
PAD = 8
SB = 64
SUB = 8
R = SB * L
CW = 4096

NC = 2
NS = 16
NW = NC * NS
ITEMS = B * L
IPW = ITEMS // NW
C = 16
NCHUNK = IPW // C


def _sc_pool_body(seq_hbm, table_hbm, out_hbm, seqv, idx0, idx1,
                  rows0, rows1, outv, sem0, sem1):
    wid = lax.axis_index("s") * NC + lax.axis_index("c")
    ibase0 = wid * IPW

    def prep_fire(t, idxv, rowsv, sem):
        fbase = (ibase0 + t * C) * PAD
        pltpu.sync_copy(seq_hbm.at[pl.ds(fbase, C * PAD)], seqv)
        srow = (ibase0 // L + t) * N_PER
        for i in range(C * PAD // 16):
            v = seqv[pl.ds(i * 16, 16)]
            idxv[pl.ds(i * 16, 16)] = jnp.minimum(v, N_PER - 1) + srow
        pltpu.async_copy(table_hbm.at[idxv], rowsv, sem)

    def drain_reduce_store(t, idxv, rowsv, sem):
        pltpu.make_async_copy(table_hbm.at[idxv], rowsv, sem).wait()

        def red(i, c2):
            for h in range(H // 16):
                acc = rowsv[i * PAD, pl.ds(h * 16, 16)]
                for p in range(1, PAD):
                    acc = acc + rowsv[i * PAD + p, pl.ds(h * 16, 16)]
                outv[i, pl.ds(h * 16, 16)] = acc
            return c2

        lax.fori_loop(0, C, red, 0)
        pltpu.sync_copy(outv, out_hbm.at[pl.ds(ibase0 + t * C, C)])

    prep_fire(0, idx0, rows0, sem0)
    prep_fire(1, idx1, rows1, sem1)

    def pair(tp, c):
        t2 = tp * 2
        for b, idxv, rowsv, sem in ((0, idx0, rows0, sem0),
                                    (1, idx1, rows1, sem1)):
            tt = t2 + b
            drain_reduce_store(tt, idxv, rowsv, sem)

            @pl.when(tt + 2 < NCHUNK)
            def _():
                prep_fire(tt + 2, idxv, rowsv, sem)

        return c

    lax.fori_loop(0, NCHUNK // 2, pair, 0)


def _sc_pool(sequence_flat, node_embedding):
    mesh = plsc.VectorSubcoreMesh(core_axis_name="c", subcore_axis_name="s",
                                  num_cores=NC, num_subcores=NS)
    f = pl.kernel(
        _sc_pool_body,
        out_type=jax.ShapeDtypeStruct((ITEMS, H), jnp.float32),
        mesh=mesh,
        scratch_types=[
            pltpu.VMEM((C * PAD,), jnp.int32),
            pltpu.VMEM((C * PAD,), jnp.int32),
            pltpu.VMEM((C * PAD,), jnp.int32),
            pltpu.VMEM((C * PAD, H), jnp.float32),
            pltpu.VMEM((C * PAD, H), jnp.float32),
            pltpu.VMEM((C, H), jnp.float32),
            pltpu.SemaphoreType.DMA,
            pltpu.SemaphoreType.DMA,
        ],
    )
    return f(sequence_flat, node_embedding)


def _attn_kernel(scsum_ref, seq_ref, il_ref, n31_ref, w1_ref, b1_ref,
                 w2_ref, b2_ref, q_ref, qb_ref, w3_ref, b3_ref, sh_ref):
    il = il_ref[...]
    seq = seq_ref[...]

    kf = jnp.sum((seq >= N_PER).astype(jnp.float32), axis=1, keepdims=True)

    rep = ((lax.broadcasted_iota(jnp.int32, (R, SB), 0) >> 4) ==
           lax.broadcasted_iota(jnp.int32, (R, SB), 1)).astype(jnp.float32)
    v31r = jnp.dot(rep, n31_ref[...], preferred_element_type=jnp.float32)
    sess_sum = scsum_ref[...] - kf * v31r
    sess = sess_sum / il

    sel_r = lax.broadcasted_iota(jnp.int32, (SB, R), 1)
    sel_s = lax.broadcasted_iota(jnp.int32, (SB, R), 0) * L + (L - 1)
    sel = (sel_r == sel_s).astype(jnp.float32)
    v_n = jnp.dot(sel, sess, preferred_element_type=jnp.float32)

    u1 = lax.dot_general(v_n, w1_ref[...], (((1,), (1,)), ((), ())),
                         preferred_element_type=jnp.float32) + b1_ref[...]
    t1 = jnp.dot(rep, u1, preferred_element_type=jnp.float32)
    t2 = lax.dot_general(sess, w2_ref[...], (((1,), (1,)), ((), ())),
                         preferred_element_type=jnp.float32) + b2_ref[...]
    a = jax.nn.sigmoid(t1 + t2)
    alpha_b = lax.dot_general(a, q_ref[...], (((1,), (0,)), ((), ())),
                              preferred_element_type=jnp.float32) + qb_ref[...]
    s_g = lax.dot_general(rep, alpha_b * sess, (((0,), (0,)), ((), ())),
                          preferred_element_type=jnp.float32)
    w3 = w3_ref[...]
    s_h = (lax.dot_general(v_n, w3[:, :H], (((1,), (1,)), ((), ())),
                           preferred_element_type=jnp.float32) +
           lax.dot_general(s_g, w3[:, H:], (((1,), (1,)), ((), ())),
                           preferred_element_type=jnp.float32) + b3_ref[...])
    sh_ref[...] = s_h


def _score_kernel(sh_ref, e_ref, cue_ref, out_ref, y_ref):
    j = pl.program_id(0)
    sh = sh_ref[...]
    eb = e_ref[...]
    scores = lax.dot_general(sh, eb, (((1,), (1,)), ((), ())),
                             preferred_element_type=jnp.float32)
    out_ref[...] = scores
    col = lax.broadcasted_iota(jnp.int32, (B, CW), 1) + j * CW
    hit = jnp.where(col == cue_ref[...], scores, 0.0)
    y_part = jnp.sum(hit, axis=1, keepdims=True)

    @pl.when(j == 0)
    def _():
        y_ref[...] = jnp.zeros_like(y_ref)

    y_ref[...] += y_part


def kernel(node_embedding, embedding_table_weight, batch, sequence, itemset_len,
           sequence_len, cue, W1_w, W1_b, W2_w, W2_b, q_w, q_b, W3_w, W3_b):
    vocab = embedding_table_weight.shape[0]
    il_f = itemset_len.astype(jnp.float32).reshape(B * L, 1)
    cue2 = cue.reshape(B, 1)
    n31 = node_embedding.reshape(B, N_PER, H)[:, N_PER - 1, :]

    sc_sum = _sc_pool(sequence.reshape(-1), node_embedding)

    n_blocks = B // SB
    s_h = pl.pallas_call(
        _attn_kernel,
        grid=(n_blocks,),
        in_specs=[
            pl.BlockSpec((R, H), lambda i: (i, 0)),
            pl.BlockSpec((R, PAD), lambda i: (i, 0)),
            pl.BlockSpec((R, 1), lambda i: (i, 0)),
            pl.BlockSpec((SB, H), lambda i: (i, 0)),
            pl.BlockSpec((H, H), lambda i: (0, 0)),
            pl.BlockSpec((1, H), lambda i: (0, 0)),
            pl.BlockSpec((H, H), lambda i: (0, 0)),
            pl.BlockSpec((1, H), lambda i: (0, 0)),
            pl.BlockSpec((H, H), lambda i: (0, 0)),
            pl.BlockSpec((1, H), lambda i: (0, 0)),
            pl.BlockSpec((H, 2 * H), lambda i: (0, 0)),
            pl.BlockSpec((1, H), lambda i: (0, 0)),
        ],
        out_specs=pl.BlockSpec((SB, H), lambda i: (i, 0)),
        out_shape=jax.ShapeDtypeStruct((B, H), jnp.float32),
    )(sc_sum, sequence, il_f, n31, W1_w, W1_b.reshape(1, H), W2_w,
      W2_b.reshape(1, H), jnp.broadcast_to(q_w.reshape(H, 1), (H, H)),
      jnp.broadcast_to(q_b.reshape(1, 1), (1, H)), W3_w, W3_b.reshape(1, H))

    n_vblocks = pl.cdiv(vocab, CW)
    all_scores, y_hat = pl.pallas_call(
        _score_kernel,
        grid=(n_vblocks,),
        in_specs=[
            pl.BlockSpec((B, H), lambda j: (0, 0)),
            pl.BlockSpec((CW, H), lambda j: (j, 0)),
            pl.BlockSpec((B, 1), lambda j: (0, 0)),
        ],
        out_specs=[
            pl.BlockSpec((B, CW), lambda j: (0, j)),
            pl.BlockSpec((B, 1), lambda j: (0, 0)),
        ],
        out_shape=[
            jax.ShapeDtypeStruct((B, vocab), jnp.float32),
            jax.ShapeDtypeStruct((B, 1), jnp.float32),
        ],
    )(s_h, embedding_table_weight, cue2)

    return (y_hat.reshape(B), all_scores)


# speedup vs baseline: 1.0294x; 1.0294x over previous
import functools

import jax
import jax.numpy as jnp
from jax import lax
from jax.experimental import pallas as pl
from jax.experimental.pallas import tpu as pltpu
from jax.experimental.pallas import tpu_sc as plsc

B = 1024
N_PER = 32
H = 128
L = 16
PAD = 8
SB = 64
SUB = 8
R = SB * L
CW = 4096

NC = 2
NS = 16
NW = NC * NS
ITEMS = B * L
IPW = ITEMS // NW
C = 16
NCHUNK = IPW // C


def _sc_pool_body(seq_hbm, table_hbm, out_hbm, seqv, idx0, idx1,
                  rows0, rows1, outv, sem0, sem1):
    wid = lax.axis_index("s") * NC + lax.axis_index("c")
    ibase0 = wid * IPW

    def prep_fire(t, idxv, rowsv, sem):
        fbase = (ibase0 + t * C) * PAD
        pltpu.sync_copy(seq_hbm.at[pl.ds(fbase, C * PAD)], seqv)
        srow = (ibase0 // L + t) * N_PER
        for i in range(C * PAD // 16):
            v = seqv[pl.ds(i * 16, 16)]
            idxv[pl.ds(i * 16, 16)] = jnp.minimum(v, N_PER - 1) + srow
        pltpu.async_copy(table_hbm.at[idxv], rowsv, sem)

    def drain_reduce_store(t, idxv, rowsv, sem):
        pltpu.make_async_copy(table_hbm.at[idxv], rowsv, sem).wait()

        @plsc.parallel_loop(0, C, unroll=2)
        def _red(i):
            for h in range(H // 16):
                sl = pl.ds(h * 16, 16)
                r = i * PAD
                acc = (((rowsv[r, sl] + rowsv[r + 1, sl]) +
                        (rowsv[r + 2, sl] + rowsv[r + 3, sl])) +
                       ((rowsv[r + 4, sl] + rowsv[r + 5, sl]) +
                        (rowsv[r + 6, sl] + rowsv[r + 7, sl])))
                outv[i, sl] = acc

        pltpu.sync_copy(outv, out_hbm.at[pl.ds(ibase0 + t * C, C)])

    prep_fire(0, idx0, rows0, sem0)
    prep_fire(1, idx1, rows1, sem1)

    def pair(tp, c):
        t2 = tp * 2
        for b, idxv, rowsv, sem in ((0, idx0, rows0, sem0),
                                    (1, idx1, rows1, sem1)):
            tt = t2 + b
            drain_reduce_store(tt, idxv, rowsv, sem)

            @pl.when(tt + 2 < NCHUNK)
            def _():
                prep_fire(tt + 2, idxv, rowsv, sem)

        return c

    lax.fori_loop(0, NCHUNK // 2, pair, 0)


def _sc_pool(sequence_flat, node_embedding):
    mesh = plsc.VectorSubcoreMesh(core_axis_name="c", subcore_axis_name="s",
                                  num_cores=NC, num_subcores=NS)
    f = pl.kernel(
        _sc_pool_body,
        out_type=jax.ShapeDtypeStruct((ITEMS, H), jnp.float32),
        mesh=mesh,
        scratch_types=[
            pltpu.VMEM((C * PAD,), jnp.int32),
            pltpu.VMEM((C * PAD,), jnp.int32),
            pltpu.VMEM((C * PAD,), jnp.int32),
            pltpu.VMEM((C * PAD, H), jnp.float32),
            pltpu.VMEM((C * PAD, H), jnp.float32),
            pltpu.VMEM((C, H), jnp.float32),
            pltpu.SemaphoreType.DMA,
            pltpu.SemaphoreType.DMA,
        ],
    )
    return f(sequence_flat, node_embedding)


def _attn_kernel(scsum_ref, seq_ref, il_ref, n31_ref, w1_ref, b1_ref,
                 w2_ref, b2_ref, q_ref, qb_ref, w3_ref, b3_ref, sh_ref):
    il = il_ref[...]
    seq = seq_ref[...]

    kf = jnp.sum((seq >= N_PER).astype(jnp.float32), axis=1, keepdims=True)

    rep = ((lax.broadcasted_iota(jnp.int32, (R, SB), 0) >> 4) ==
           lax.broadcasted_iota(jnp.int32, (R, SB), 1)).astype(jnp.float32)
    v31r = jnp.dot(rep, n31_ref[...], preferred_element_type=jnp.float32)
    sess_sum = scsum_ref[...] - kf * v31r
    sess = sess_sum / il

    sel_r = lax.broadcasted_iota(jnp.int32, (SB, R), 1)
    sel_s = lax.broadcasted_iota(jnp.int32, (SB, R), 0) * L + (L - 1)
    sel = (sel_r == sel_s).astype(jnp.float32)
    v_n = jnp.dot(sel, sess, preferred_element_type=jnp.float32)

    u1 = lax.dot_general(v_n, w1_ref[...], (((1,), (1,)), ((), ())),
                         preferred_element_type=jnp.float32) + b1_ref[...]
    t1 = jnp.dot(rep, u1, preferred_element_type=jnp.float32)
    t2 = lax.dot_general(sess, w2_ref[...], (((1,), (1,)), ((), ())),
                         preferred_element_type=jnp.float32) + b2_ref[...]
    a = jax.nn.sigmoid(t1 + t2)
    alpha_b = lax.dot_general(a, q_ref[...], (((1,), (0,)), ((), ())),
                              preferred_element_type=jnp.float32) + qb_ref[...]
    s_g = lax.dot_general(rep, alpha_b * sess, (((0,), (0,)), ((), ())),
                          preferred_element_type=jnp.float32)
    w3 = w3_ref[...]
    s_h = (lax.dot_general(v_n, w3[:, :H], (((1,), (1,)), ((), ())),
                           preferred_element_type=jnp.float32) +
           lax.dot_general(s_g, w3[:, H:], (((1,), (1,)), ((), ())),
                           preferred_element_type=jnp.float32) + b3_ref[...])
    sh_ref[...] = s_h


def _score_kernel(sh_ref, e_ref, cue_ref, out_ref, y_ref):
    j = pl.program_id(0)
    sh = sh_ref[...]
    eb = e_ref[...]
    scores = lax.dot_general(sh, eb, (((1,), (1,)), ((), ())),
                             preferred_element_type=jnp.float32)
    out_ref[...] = scores
    col = lax.broadcasted_iota(jnp.int32, (B, CW), 1) + j * CW
    hit = jnp.where(col == cue_ref[...], scores, 0.0)
    y_part = jnp.sum(hit, axis=1, keepdims=True)

    @pl.when(j == 0)
    def _():
        y_ref[...] = jnp.zeros_like(y_ref)

    y_ref[...] += y_part


def kernel(node_embedding, embedding_table_weight, batch, sequence, itemset_len,
           sequence_len, cue, W1_w, W1_b, W2_w, W2_b, q_w, q_b, W3_w, W3_b):
    vocab = embedding_table_weight.shape[0]
    il_f = itemset_len.astype(jnp.float32).reshape(B * L, 1)
    cue2 = cue.reshape(B, 1)
    n31 = node_embedding.reshape(B, N_PER, H)[:, N_PER - 1, :]

    sc_sum = _sc_pool(sequence.reshape(-1), node_embedding)

    n_blocks = B // SB
    s_h = pl.pallas_call(
        _attn_kernel,
        grid=(n_blocks,),
        in_specs=[
            pl.BlockSpec((R, H), lambda i: (i, 0)),
            pl.BlockSpec((R, PAD), lambda i: (i, 0)),
            pl.BlockSpec((R, 1), lambda i: (i, 0)),
            pl.BlockSpec((SB, H), lambda i: (i, 0)),
            pl.BlockSpec((H, H), lambda i: (0, 0)),
            pl.BlockSpec((1, H), lambda i: (0, 0)),
            pl.BlockSpec((H, H), lambda i: (0, 0)),
            pl.BlockSpec((1, H), lambda i: (0, 0)),
            pl.BlockSpec((H, H), lambda i: (0, 0)),
            pl.BlockSpec((1, H), lambda i: (0, 0)),
            pl.BlockSpec((H, 2 * H), lambda i: (0, 0)),
            pl.BlockSpec((1, H), lambda i: (0, 0)),
        ],
        out_specs=pl.BlockSpec((SB, H), lambda i: (i, 0)),
        out_shape=jax.ShapeDtypeStruct((B, H), jnp.float32),
    )(sc_sum, sequence, il_f, n31, W1_w, W1_b.reshape(1, H), W2_w,
      W2_b.reshape(1, H), jnp.broadcast_to(q_w.reshape(H, 1), (H, H)),
      jnp.broadcast_to(q_b.reshape(1, 1), (1, H)), W3_w, W3_b.reshape(1, H))

    n_vblocks = pl.cdiv(vocab, CW)
    all_scores, y_hat = pl.pallas_call(
        _score_kernel,
        grid=(n_vblocks,),
        in_specs=[
            pl.BlockSpec((B, H), lambda j: (0, 0)),
            pl.BlockSpec((CW, H), lambda j: (j, 0)),
            pl.BlockSpec((B, 1), lambda j: (0, 0)),
        ],
        out_specs=[
            pl.BlockSpec((B, CW), lambda j: (0, j)),
            pl.BlockSpec((B, 1), lambda j: (0, 0)),
        ],
        out_shape=[
            jax.ShapeDtypeStruct((B, vocab), jnp.float32),
            jax.ShapeDtypeStruct((B, 1), jnp.float32),
        ],
    )(s_h, embedding_table_weight, cue2)

    return (y_hat.reshape(B), all_scores)


# speedup vs baseline: 1.0462x; 1.0163x over previous
import jax
import jax.numpy as jnp
from jax import lax
from jax.experimental import pallas as pl
from jax.experimental.pallas import tpu as pltpu
from jax.experimental.pallas import tpu_sc as plsc

B = 1024
HALF = 512
N_PER = 32
H = 128
L = 16
PAD = 8
SB = 64
SUB = 8
R = SB * L
RS = SUB * L
NSR = SUB * N_PER
CW = 4096

NC = 2
NS = 16
NW = NC * NS
SC_ITEMS = HALF * L
SC_BASE = HALF * L
IPW = SC_ITEMS // NW
C = 16
NCHUNK = IPW // C


def _sc_pool_body(seq_hbm, table_hbm, out_hbm, seqv, idx0, idx1,
                  rows0, rows1, outv, sem0, sem1):
    wid = lax.axis_index("s") * NC + lax.axis_index("c")
    ibase0 = SC_BASE + wid * IPW
    obase0 = wid * IPW

    def prep_fire(t, idxv, rowsv, sem):
        fbase = (ibase0 + t * C) * PAD
        pltpu.sync_copy(seq_hbm.at[pl.ds(fbase, C * PAD)], seqv)
        srow = (ibase0 // L + t) * N_PER
        for i in range(C * PAD // 16):
            v = seqv[pl.ds(i * 16, 16)]
            idxv[pl.ds(i * 16, 16)] = jnp.minimum(v, N_PER - 1) + srow
        pltpu.async_copy(table_hbm.at[idxv], rowsv, sem)

    def drain_reduce_store(t, idxv, rowsv, sem):
        pltpu.make_async_copy(table_hbm.at[idxv], rowsv, sem).wait()

        @plsc.parallel_loop(0, C, unroll=2)
        def _red(i):
            for h in range(H // 16):
                sl = pl.ds(h * 16, 16)
                r = i * PAD
                acc = (((rowsv[r, sl] + rowsv[r + 1, sl]) +
                        (rowsv[r + 2, sl] + rowsv[r + 3, sl])) +
                       ((rowsv[r + 4, sl] + rowsv[r + 5, sl]) +
                        (rowsv[r + 6, sl] + rowsv[r + 7, sl])))
                outv[i, sl] = acc

        pltpu.sync_copy(outv, out_hbm.at[pl.ds(obase0 + t * C, C)])

    prep_fire(0, idx0, rows0, sem0)
    prep_fire(1, idx1, rows1, sem1)

    def pair(tp, c):
        t2 = tp * 2
        for b, idxv, rowsv, sem in ((0, idx0, rows0, sem0),
                                    (1, idx1, rows1, sem1)):
            tt = t2 + b
            drain_reduce_store(tt, idxv, rowsv, sem)

            @pl.when(tt + 2 < NCHUNK)
            def _():
                prep_fire(tt + 2, idxv, rowsv, sem)

        return c

    lax.fori_loop(0, NCHUNK // 2, pair, 0)


def _sc_pool(sequence_flat, node_embedding):
    mesh = plsc.VectorSubcoreMesh(core_axis_name="c", subcore_axis_name="s",
                                  num_cores=NC, num_subcores=NS)
    f = pl.kernel(
        _sc_pool_body,
        out_type=jax.ShapeDtypeStruct((SC_ITEMS, H), jnp.float32),
        mesh=mesh,
        scratch_types=[
            pltpu.VMEM((C * PAD,), jnp.int32),
            pltpu.VMEM((C * PAD,), jnp.int32),
            pltpu.VMEM((C * PAD,), jnp.int32),
            pltpu.VMEM((C * PAD, H), jnp.float32),
            pltpu.VMEM((C * PAD, H), jnp.float32),
            pltpu.VMEM((C, H), jnp.float32),
            pltpu.SemaphoreType.DMA,
            pltpu.SemaphoreType.DMA,
        ],
    )
    return f(sequence_flat, node_embedding)


def _attn_tail(sess, il, w1_ref, b1_ref, w2_ref, b2_ref, q_ref, qb_ref,
               w3_ref, b3_ref, rep):
    sel_r = lax.broadcasted_iota(jnp.int32, (SB, R), 1)
    sel_s = lax.broadcasted_iota(jnp.int32, (SB, R), 0) * L + (L - 1)
    sel = (sel_r == sel_s).astype(jnp.float32)
    v_n = jnp.dot(sel, sess, preferred_element_type=jnp.float32)

    u1 = lax.dot_general(v_n, w1_ref[...], (((1,), (1,)), ((), ())),
                         preferred_element_type=jnp.float32) + b1_ref[...]
    t1 = jnp.dot(rep, u1, preferred_element_type=jnp.float32)
    t2 = lax.dot_general(sess, w2_ref[...], (((1,), (1,)), ((), ())),
                         preferred_element_type=jnp.float32) + b2_ref[...]
    a = jax.nn.sigmoid(t1 + t2)
    alpha_b = lax.dot_general(a, q_ref[...], (((1,), (0,)), ((), ())),
                              preferred_element_type=jnp.float32) + qb_ref[...]
    s_g = lax.dot_general(rep, alpha_b * sess, (((0,), (0,)), ((), ())),
                          preferred_element_type=jnp.float32)
    w3 = w3_ref[...]
    return (lax.dot_general(v_n, w3[:, :H], (((1,), (1,)), ((), ())),
                            preferred_element_type=jnp.float32) +
            lax.dot_general(s_g, w3[:, H:], (((1,), (1,)), ((), ())),
                            preferred_element_type=jnp.float32) + b3_ref[...])


def _rep():
    return ((lax.broadcasted_iota(jnp.int32, (R, SB), 0) >> 4) ==
            lax.broadcasted_iota(jnp.int32, (R, SB), 1)).astype(jnp.float32)


def _attn1_kernel(nodes_ref, seq_ref, il_ref, w1_ref, b1_ref, w2_ref, b2_ref,
                  q_ref, qb_ref, w3_ref, b3_ref, sh_ref):
    il = il_ref[...]
    base = (lax.broadcasted_iota(jnp.int32, (RS, NSR), 0) >> 4) << 5
    cols = lax.broadcasted_iota(jnp.int32, (RS, NSR), 1)
    parts = []
    for g in range(SB // SUB):
        seq_g = seq_ref[g * RS:(g + 1) * RS, :]
        counts = jnp.zeros((RS, NSR), jnp.float32)
        for p in range(PAD):
            sp = seq_g[:, p:p + 1]
            t = jnp.where(sp < N_PER, sp, 100000) + base
            counts = counts + (cols == t).astype(jnp.float32)
        parts.append(jnp.dot(counts, nodes_ref[g * NSR:(g + 1) * NSR, :],
                             preferred_element_type=jnp.float32))
    sess = jnp.concatenate(parts, axis=0) / il
    sh_ref[...] = _attn_tail(sess, il, w1_ref, b1_ref, w2_ref, b2_ref,
                             q_ref, qb_ref, w3_ref, b3_ref, _rep())


def _attn2_kernel(scsum_ref, seq_ref, il_ref, n31_ref, w1_ref, b1_ref,
                  w2_ref, b2_ref, q_ref, qb_ref, w3_ref, b3_ref, sh_ref):
    il = il_ref[...]
    seq = seq_ref[...]
    kf = jnp.sum((seq >= N_PER).astype(jnp.float32), axis=1, keepdims=True)
    rep = _rep()
    v31r = jnp.dot(rep, n31_ref[...], preferred_element_type=jnp.float32)
    sess = (scsum_ref[...] - kf * v31r) / il
    sh_ref[...] = _attn_tail(sess, il, w1_ref, b1_ref, w2_ref, b2_ref,
                             q_ref, qb_ref, w3_ref, b3_ref, rep)


def _score_kernel(sh_ref, e_ref, cue_ref, out_ref, y_ref):
    j = pl.program_id(0)
    sh = sh_ref[...]
    eb = e_ref[...]
    scores = lax.dot_general(sh, eb, (((1,), (1,)), ((), ())),
                             preferred_element_type=jnp.float32)
    out_ref[...] = scores
    col = lax.broadcasted_iota(jnp.int32, (B, CW), 1) + j * CW
    hit = jnp.where(col == cue_ref[...], scores, 0.0)
    y_part = jnp.sum(hit, axis=1, keepdims=True)

    @pl.when(j == 0)
    def _():
        y_ref[...] = jnp.zeros_like(y_ref)

    y_ref[...] += y_part


def _weight_specs():
    return [
        pl.BlockSpec((H, H), lambda i: (0, 0)),
        pl.BlockSpec((1, H), lambda i: (0, 0)),
        pl.BlockSpec((H, H), lambda i: (0, 0)),
        pl.BlockSpec((1, H), lambda i: (0, 0)),
        pl.BlockSpec((H, H), lambda i: (0, 0)),
        pl.BlockSpec((1, H), lambda i: (0, 0)),
        pl.BlockSpec((H, 2 * H), lambda i: (0, 0)),
        pl.BlockSpec((1, H), lambda i: (0, 0)),
    ]


def kernel(node_embedding, embedding_table_weight, batch, sequence, itemset_len,
           sequence_len, cue, W1_w, W1_b, W2_w, W2_b, q_w, q_b, W3_w, W3_b):
    vocab = embedding_table_weight.shape[0]
    il_f = itemset_len.astype(jnp.float32).reshape(B * L, 1)
    cue2 = cue.reshape(B, 1)
    weights = (W1_w, W1_b.reshape(1, H), W2_w, W2_b.reshape(1, H),
               jnp.broadcast_to(q_w.reshape(H, 1), (H, H)),
               jnp.broadcast_to(q_b.reshape(1, 1), (1, H)),
               W3_w, W3_b.reshape(1, H))

    sc_sum = _sc_pool(sequence.reshape(-1), node_embedding)

    nb = HALF // SB
    s_h1 = pl.pallas_call(
        _attn1_kernel,
        grid=(nb,),
        in_specs=[
            pl.BlockSpec((SB * N_PER, H), lambda i: (i, 0)),
            pl.BlockSpec((R, PAD), lambda i: (i, 0)),
            pl.BlockSpec((R, 1), lambda i: (i, 0)),
        ] + _weight_specs(),
        out_specs=pl.BlockSpec((SB, H), lambda i: (i, 0)),
        out_shape=jax.ShapeDtypeStruct((HALF, H), jnp.float32),
    )(node_embedding[:HALF * N_PER], sequence[:HALF * L], il_f[:HALF * L],
      *weights)

    n31 = node_embedding.reshape(B, N_PER, H)[HALF:, N_PER - 1, :]
    s_h2 = pl.pallas_call(
        _attn2_kernel,
        grid=(nb,),
        in_specs=[
            pl.BlockSpec((R, H), lambda i: (i, 0)),
            pl.BlockSpec((R, PAD), lambda i: (i, 0)),
            pl.BlockSpec((R, 1), lambda i: (i, 0)),
            pl.BlockSpec((SB, H), lambda i: (i, 0)),
        ] + _weight_specs(),
        out_specs=pl.BlockSpec((SB, H), lambda i: (i, 0)),
        out_shape=jax.ShapeDtypeStruct((HALF, H), jnp.float32),
    )(sc_sum, sequence[HALF * L:], il_f[HALF * L:], n31, *weights)

    s_h = jnp.concatenate([s_h1, s_h2], axis=0)

    n_vblocks = pl.cdiv(vocab, CW)
    all_scores, y_hat = pl.pallas_call(
        _score_kernel,
        grid=(n_vblocks,),
        in_specs=[
            pl.BlockSpec((B, H), lambda j: (0, 0)),
            pl.BlockSpec((CW, H), lambda j: (j, 0)),
            pl.BlockSpec((B, 1), lambda j: (0, 0)),
        ],
        out_specs=[
            pl.BlockSpec((B, CW), lambda j: (0, j)),
            pl.BlockSpec((B, 1), lambda j: (0, 0)),
        ],
        out_shape=[
            jax.ShapeDtypeStruct((B, vocab), jnp.float32),
            jax.ShapeDtypeStruct((B, 1), jnp.float32),
        ],
    )(s_h, embedding_table_weight, cue2)

    return (y_hat.reshape(B), all_scores)


# speedup vs baseline: 1.0496x; 1.0033x over previous
import jax
import jax.numpy as jnp
from jax import lax
from jax.experimental import pallas as pl
from jax.experimental.pallas import tpu as pltpu
from jax.experimental.pallas import tpu_sc as plsc

B = 1024
HALF = 512
N_PER = 32
H = 128
L = 16
PAD = 8
SB = 64
SUB = 8
R = SB * L
RS = SUB * L
NSR = SUB * N_PER
CW = 4096

NC = 2
NS = 16
NW = NC * NS
SC_ITEMS = HALF * L
SC_BASE = HALF * L
IPW = SC_ITEMS // NW
C = 16
NCHUNK = IPW // C
NBUF = 4


def _sc_pool_body(seq_hbm, table_hbm, out_hbm, seqv,
                  idx0, idx1, idx2, idx3,
                  rows0, rows1, rows2, rows3,
                  out0, out1, out2, out3,
                  sem0, sem1, sem2, sem3,
                  osem0, osem1, osem2, osem3):
    wid = lax.axis_index("s") * NC + lax.axis_index("c")
    ibase0 = SC_BASE + wid * IPW
    obase0 = wid * IPW

    pltpu.sync_copy(seq_hbm.at[pl.ds(ibase0 * PAD, IPW * PAD)], seqv)

    idxs = (idx0, idx1, idx2, idx3)
    rows = (rows0, rows1, rows2, rows3)
    outs = (out0, out1, out2, out3)
    sems = (sem0, sem1, sem2, sem3)
    osems = (osem0, osem1, osem2, osem3)

    def prep_fire(t, idxv, rowsv, sem):
        srow = (ibase0 // L + t) * N_PER
        foff = t * C * PAD
        for i in range(C * PAD // 16):
            v = seqv[pl.ds(foff + i * 16, 16)]
            idxv[pl.ds(i * 16, 16)] = jnp.minimum(v, N_PER - 1) + srow
        pltpu.async_copy(table_hbm.at[idxv], rowsv, sem)

    def drain_reduce_store(t, idxv, rowsv, outv, sem, osem):
        pltpu.make_async_copy(table_hbm.at[idxv], rowsv, sem).wait()

        @pl.when(t >= NBUF)
        def _():
            pltpu.make_async_copy(
                outv, out_hbm.at[pl.ds(obase0 + (t - NBUF) * C, C)],
                osem).wait()

        @plsc.parallel_loop(0, C, unroll=2)
        def _red(i):
            for h in range(H // 16):
                sl = pl.ds(h * 16, 16)
                r = i * PAD
                acc = (((rowsv[r, sl] + rowsv[r + 1, sl]) +
                        (rowsv[r + 2, sl] + rowsv[r + 3, sl])) +
                       ((rowsv[r + 4, sl] + rowsv[r + 5, sl]) +
                        (rowsv[r + 6, sl] + rowsv[r + 7, sl])))
                outv[i, sl] = acc

        pltpu.async_copy(outv, out_hbm.at[pl.ds(obase0 + t * C, C)], osem)

    for b in range(NBUF):
        prep_fire(b, idxs[b], rows[b], sems[b])

    def quad(tp, c):
        t4 = tp * NBUF
        for b in range(NBUF):
            tt = t4 + b
            drain_reduce_store(tt, idxs[b], rows[b], outs[b], sems[b],
                               osems[b])

            @pl.when(tt + NBUF < NCHUNK)
            def _():
                prep_fire(tt + NBUF, idxs[b], rows[b], sems[b])

        return c

    lax.fori_loop(0, NCHUNK // NBUF, quad, 0)

    for b in range(NBUF):
        t_last = NCHUNK - NBUF + b
        pltpu.make_async_copy(
            outs[b], out_hbm.at[pl.ds(obase0 + t_last * C, C)],
            osems[b]).wait()


def _sc_pool(sequence_flat, node_embedding):
    mesh = plsc.VectorSubcoreMesh(core_axis_name="c", subcore_axis_name="s",
                                  num_cores=NC, num_subcores=NS)
    f = pl.kernel(
        _sc_pool_body,
        out_type=jax.ShapeDtypeStruct((SC_ITEMS, H), jnp.float32),
        mesh=mesh,
        scratch_types=(
            [pltpu.VMEM((IPW * PAD,), jnp.int32)]
            + [pltpu.VMEM((C * PAD,), jnp.int32) for _ in range(NBUF)]
            + [pltpu.VMEM((C * PAD, H), jnp.float32) for _ in range(NBUF)]
            + [pltpu.VMEM((C, H), jnp.float32) for _ in range(NBUF)]
            + [pltpu.SemaphoreType.DMA for _ in range(2 * NBUF)]
        ),
    )
    return f(sequence_flat, node_embedding)


def _attn_tail(sess, il, w1_ref, b1_ref, w2_ref, b2_ref, q_ref, qb_ref,
               w3_ref, b3_ref, rep):
    sel_r = lax.broadcasted_iota(jnp.int32, (SB, R), 1)
    sel_s = lax.broadcasted_iota(jnp.int32, (SB, R), 0) * L + (L - 1)
    sel = (sel_r == sel_s).astype(jnp.float32)
    v_n = jnp.dot(sel, sess, preferred_element_type=jnp.float32)

    u1 = lax.dot_general(v_n, w1_ref[...], (((1,), (1,)), ((), ())),
                         preferred_element_type=jnp.float32) + b1_ref[...]
    t1 = jnp.dot(rep, u1, preferred_element_type=jnp.float32)
    t2 = lax.dot_general(sess, w2_ref[...], (((1,), (1,)), ((), ())),
                         preferred_element_type=jnp.float32) + b2_ref[...]
    a = jax.nn.sigmoid(t1 + t2)
    alpha_b = lax.dot_general(a, q_ref[...], (((1,), (0,)), ((), ())),
                              preferred_element_type=jnp.float32) + qb_ref[...]
    s_g = lax.dot_general(rep, alpha_b * sess, (((0,), (0,)), ((), ())),
                          preferred_element_type=jnp.float32)
    w3 = w3_ref[...]
    return (lax.dot_general(v_n, w3[:, :H], (((1,), (1,)), ((), ())),
                            preferred_element_type=jnp.float32) +
            lax.dot_general(s_g, w3[:, H:], (((1,), (1,)), ((), ())),
                            preferred_element_type=jnp.float32) + b3_ref[...])


def _rep():
    return ((lax.broadcasted_iota(jnp.int32, (R, SB), 0) >> 4) ==
            lax.broadcasted_iota(jnp.int32, (R, SB), 1)).astype(jnp.float32)


def _attn1_kernel(nodes_ref, seq_ref, il_ref, w1_ref, b1_ref, w2_ref, b2_ref,
                  q_ref, qb_ref, w3_ref, b3_ref, sh_ref):
    il = il_ref[...]
    base = (lax.broadcasted_iota(jnp.int32, (RS, NSR), 0) >> 4) << 5
    cols = lax.broadcasted_iota(jnp.int32, (RS, NSR), 1)
    parts = []
    for g in range(SB // SUB):
        seq_g = seq_ref[g * RS:(g + 1) * RS, :]
        counts = jnp.zeros((RS, NSR), jnp.float32)
        for p in range(PAD):
            sp = seq_g[:, p:p + 1]
            t = jnp.where(sp < N_PER, sp, 100000) + base
            counts = counts + (cols == t).astype(jnp.float32)
        parts.append(jnp.dot(counts, nodes_ref[g * NSR:(g + 1) * NSR, :],
                             preferred_element_type=jnp.float32))
    sess = jnp.concatenate(parts, axis=0) / il
    sh_ref[...] = _attn_tail(sess, il, w1_ref, b1_ref, w2_ref, b2_ref,
                             q_ref, qb_ref, w3_ref, b3_ref, _rep())


def _attn2_kernel(scsum_ref, seq_ref, il_ref, n31_ref, w1_ref, b1_ref,
                  w2_ref, b2_ref, q_ref, qb_ref, w3_ref, b3_ref, sh_ref):
    il = il_ref[...]
    seq = seq_ref[...]
    kf = jnp.sum((seq >= N_PER).astype(jnp.float32), axis=1, keepdims=True)
    rep = _rep()
    v31r = jnp.dot(rep, n31_ref[...], preferred_element_type=jnp.float32)
    sess = (scsum_ref[...] - kf * v31r) / il
    sh_ref[...] = _attn_tail(sess, il, w1_ref, b1_ref, w2_ref, b2_ref,
                             q_ref, qb_ref, w3_ref, b3_ref, rep)


def _score_kernel(sh_ref, e_ref, cue_ref, out_ref, y_ref):
    j = pl.program_id(0)
    sh = sh_ref[...]
    eb = e_ref[...]
    scores = lax.dot_general(sh, eb, (((1,), (1,)), ((), ())),
                             preferred_element_type=jnp.float32)
    out_ref[...] = scores
    col = lax.broadcasted_iota(jnp.int32, (B, CW), 1) + j * CW
    hit = jnp.where(col == cue_ref[...], scores, 0.0)
    y_part = jnp.sum(hit, axis=1, keepdims=True)

    @pl.when(j == 0)
    def _():
        y_ref[...] = jnp.zeros_like(y_ref)

    y_ref[...] += y_part


def _weight_specs():
    return [
        pl.BlockSpec((H, H), lambda i: (0, 0)),
        pl.BlockSpec((1, H), lambda i: (0, 0)),
        pl.BlockSpec((H, H), lambda i: (0, 0)),
        pl.BlockSpec((1, H), lambda i: (0, 0)),
        pl.BlockSpec((H, H), lambda i: (0, 0)),
        pl.BlockSpec((1, H), lambda i: (0, 0)),
        pl.BlockSpec((H, 2 * H), lambda i: (0, 0)),
        pl.BlockSpec((1, H), lambda i: (0, 0)),
    ]


def kernel(node_embedding, embedding_table_weight, batch, sequence, itemset_len,
           sequence_len, cue, W1_w, W1_b, W2_w, W2_b, q_w, q_b, W3_w, W3_b):
    vocab = embedding_table_weight.shape[0]
    il_f = itemset_len.astype(jnp.float32).reshape(B * L, 1)
    cue2 = cue.reshape(B, 1)
    weights = (W1_w, W1_b.reshape(1, H), W2_w, W2_b.reshape(1, H),
               jnp.broadcast_to(q_w.reshape(H, 1), (H, H)),
               jnp.broadcast_to(q_b.reshape(1, 1), (1, H)),
               W3_w, W3_b.reshape(1, H))

    sc_sum = _sc_pool(sequence.reshape(-1), node_embedding)

    nb = HALF // SB
    s_h1 = pl.pallas_call(
        _attn1_kernel,
        grid=(nb,),
        in_specs=[
            pl.BlockSpec((SB * N_PER, H), lambda i: (i, 0)),
            pl.BlockSpec((R, PAD), lambda i: (i, 0)),
            pl.BlockSpec((R, 1), lambda i: (i, 0)),
        ] + _weight_specs(),
        out_specs=pl.BlockSpec((SB, H), lambda i: (i, 0)),
        out_shape=jax.ShapeDtypeStruct((HALF, H), jnp.float32),
    )(node_embedding[:HALF * N_PER], sequence[:HALF * L], il_f[:HALF * L],
      *weights)

    n31 = node_embedding.reshape(B, N_PER, H)[HALF:, N_PER - 1, :]
    s_h2 = pl.pallas_call(
        _attn2_kernel,
        grid=(nb,),
        in_specs=[
            pl.BlockSpec((R, H), lambda i: (i, 0)),
            pl.BlockSpec((R, PAD), lambda i: (i, 0)),
            pl.BlockSpec((R, 1), lambda i: (i, 0)),
            pl.BlockSpec((SB, H), lambda i: (i, 0)),
        ] + _weight_specs(),
        out_specs=pl.BlockSpec((SB, H), lambda i: (i, 0)),
        out_shape=jax.ShapeDtypeStruct((HALF, H), jnp.float32),
    )(sc_sum, sequence[HALF * L:], il_f[HALF * L:], n31, *weights)

    s_h = jnp.concatenate([s_h1, s_h2], axis=0)

    n_vblocks = pl.cdiv(vocab, CW)
    all_scores, y_hat = pl.pallas_call(
        _score_kernel,
        grid=(n_vblocks,),
        in_specs=[
            pl.BlockSpec((B, H), lambda j: (0, 0)),
            pl.BlockSpec((CW, H), lambda j: (j, 0)),
            pl.BlockSpec((B, 1), lambda j: (0, 0)),
        ],
        out_specs=[
            pl.BlockSpec((B, CW), lambda j: (0, j)),
            pl.BlockSpec((B, 1), lambda j: (0, 0)),
        ],
        out_shape=[
            jax.ShapeDtypeStruct((B, vocab), jnp.float32),
            jax.ShapeDtypeStruct((B, 1), jnp.float32),
        ],
    )(s_h, embedding_table_weight, cue2)

    return (y_hat.reshape(B), all_scores)
